# baseline (device time: 3573580 ns/iter reference)
import jax
import jax.numpy as jnp
from jax import lax
from jax.experimental import pallas as pl
from jax.experimental.pallas import tpu as pltpu

W = 8
NT = 16384
S = NT // W
D = 1024
H = 1024
NE = 64
EL = NE // W
CAP = 204
CST = 256
ROWS = 512
BIG = jnp.int32(1 << 30)
PACKN = 2048 + 8 * W + ROWS
I32 = jnp.int32


def _a2a(src, offs, rows, *, cid):
    _, c = src.shape

    def body(offs_ref, src_ref, out_ref, send_sems, recv_sems):
        me = lax.axis_index("i")

        bar = pltpu.get_barrier_semaphore()
        for d in range(W):
            @pl.when(me != d)
            def _():
                pl.semaphore_signal(
                    bar, inc=1,
                    device_id=(d,), device_id_type=pl.DeviceIdType.MESH,
                )
        pl.semaphore_wait(bar, W - 1)

        for d in range(W):
            off = pl.multiple_of(offs_ref[d], 8)

            @pl.when(me == d)
            def _():
                out_ref[d] = src_ref[pl.ds(off, rows)]

            @pl.when(me != d)
            def _():
                pltpu.make_async_remote_copy(
                    src_ref=src_ref.at[pl.ds(off, rows)],
                    dst_ref=out_ref.at[me],
                    send_sem=send_sems.at[d],
                    recv_sem=recv_sems.at[me],
                    device_id=(d,),
                    device_id_type=pl.DeviceIdType.MESH,
                ).start()

        for s in range(W):
            @pl.when(me != s)
            def _():
                pltpu.make_async_remote_copy(
                    src_ref=src_ref.at[pl.ds(0, rows)],
                    dst_ref=out_ref.at[s],
                    send_sem=send_sems.at[s],
                    recv_sem=recv_sems.at[s],
                    device_id=(s,),
                    device_id_type=pl.DeviceIdType.MESH,
                ).wait_recv()

        for d in range(W):
            @pl.when(me != d)
            def _():
                pltpu.make_async_remote_copy(
                    src_ref=src_ref.at[pl.ds(0, rows)],
                    dst_ref=out_ref.at[d],
                    send_sem=send_sems.at[d],
                    recv_sem=recv_sems.at[d],
                    device_id=(d,),
                    device_id_type=pl.DeviceIdType.MESH,
                ).wait_send()

    return pl.pallas_call(
        body,
        out_shape=jax.ShapeDtypeStruct((W, rows, c), src.dtype),
        in_specs=[
            pl.BlockSpec(memory_space=pltpu.SMEM),
            pl.BlockSpec(memory_space=pltpu.VMEM),
        ],
        out_specs=pl.BlockSpec(memory_space=pltpu.VMEM),
        scratch_shapes=[
            pltpu.SemaphoreType.DMA((W,)),
            pltpu.SemaphoreType.DMA((W,)),
        ],
        compiler_params=pltpu.CompilerParams(collective_id=cid),
    )(offs, src)


def _moe_matmul(xin, expert_W):

    def body(x_ref, w_ref, o_ref):
        o_ref[...] = jnp.dot(
            x_ref[...], w_ref[0], preferred_element_type=jnp.float32
        )

    return pl.pallas_call(
        body,
        grid=(EL,),
        in_specs=[
            pl.BlockSpec((CST, D), lambda e: (e, 0)),
            pl.BlockSpec((1, D, H), lambda e: (e, 0, 0)),
        ],
        out_specs=pl.BlockSpec((CST, H), lambda e: (e, 0)),
        out_shape=jax.ShapeDtypeStruct((EL * CST, H), jnp.float32),
    )(xin, expert_W)


def _pack_gather_idx(grp):
    n = grp.shape[0]
    oh = grp[:, None] == jnp.arange(W + 1, dtype=grp.dtype)[None, :]
    within = (
        jnp.take_along_axis(
            jnp.cumsum(oh.astype(I32), axis=0), grp[:, None].astype(I32), axis=1
        )[:, 0]
        - 1
    )
    cnts = oh.sum(axis=0).astype(I32)[:W]
    acnts = ((cnts + 7) // 8) * 8
    aoffs = jnp.cumsum(acnts) - acnts
    aoffs_ext = jnp.concatenate([aoffs, jnp.array([BIG], I32)])
    pos = aoffs_ext[grp] + within
    idx = (
        jnp.zeros(PACKN, I32)
        .at[pos]
        .set(jnp.arange(n, dtype=I32), mode="drop")
    )
    return idx, aoffs


def kernel(x, router_W, route_idx, expert_W):
    del router_W
    me = lax.axis_index("i")

    rloc = route_idx.reshape(16, 128)
    route = _a2a(rloc, jnp.zeros((W,), I32), 16, cid=0).reshape(NT)

    perm = jnp.argsort(route, stable=True)
    sorted_e = route[perm]
    starts = jnp.searchsorted(sorted_e, jnp.arange(NE, dtype=sorted_e.dtype))
    rank_sorted = jnp.arange(NT, dtype=I32) - starts[sorted_e].astype(I32)
    rank = jnp.zeros(NT, I32).at[perm].set(rank_sorted)
    keep = rank < CAP
    gslot = jnp.where(keep, route * CST + rank, BIG)
    tok_of_gslot = (
        jnp.full(NE * CST, -1, I32)
        .at[gslot]
        .set(jnp.arange(NT, dtype=I32), mode="drop")
    )

    myroute = route_idx[:, 0]
    mykeep = lax.dynamic_slice(keep, (me * S,), (S,))
    dest = jnp.where(mykeep, myroute // EL, W).astype(I32)
    pidx, offs = _pack_gather_idx(dest)
    x_pack = x[pidx]

    xrecv = _a2a(x_pack, offs, ROWS, cid=1)

    keep2 = keep.reshape(W, S)
    route2 = route.reshape(W, S)
    rank2 = rank.reshape(W, S)
    tome = keep2 & ((route2 // EL) == me)
    lslot = jnp.where(tome, (route2 - me * EL) * CST + rank2, BIG)
    j2 = jnp.cumsum(tome.astype(I32), axis=1) - 1
    src_row = jnp.arange(W, dtype=I32)[:, None] * ROWS + j2
    inv2 = (
        jnp.full(EL * CST, BIG, I32)
        .at[lslot.reshape(-1)]
        .set(src_row.reshape(-1), mode="drop")
    )
    xin = jnp.take(
        xrecv.reshape(W * ROWS, D), inv2, axis=0, mode="fill", fill_value=0.0
    )

    y = _moe_matmul(xin, expert_W)

    mytoks = lax.dynamic_slice(tok_of_gslot, (me * EL * CST,), (EL * CST,))
    cdest = jnp.where(mytoks >= 0, mytoks // S, W).astype(I32)
    cidx, coffs = _pack_gather_idx(cdest)
    y_pack = y[cidx]

    yrecv = _a2a(y_pack, coffs, ROWS, cid=2)

    toks_by_s = tok_of_gslot.reshape(W, EL * CST)
    mine = (toks_by_s >= me * S) & (toks_by_s < (me + 1) * S)
    mytok = jnp.where(mine, toks_by_s - me * S, BIG)
    j3 = jnp.cumsum(mine.astype(I32), axis=1) - 1
    src_row3 = jnp.arange(W, dtype=I32)[:, None] * ROWS + j3
    inv4 = (
        jnp.full(S, BIG, I32)
        .at[mytok.reshape(-1)]
        .set(src_row3.reshape(-1), mode="drop")
    )
    out = jnp.take(
        yrecv.reshape(W * ROWS, H), inv4, axis=0, mode="fill", fill_value=0.0
    )
    return out


# device time: 739255 ns/iter; 4.8340x vs baseline; 4.8340x over previous
import jax
import jax.numpy as jnp
from jax import lax
from jax.experimental import pallas as pl
from jax.experimental.pallas import tpu as pltpu

W = 8
NT = 16384
S = NT // W
D = 1024
H = 1024
NE = 64
EL = NE // W
CAP = 204
CST = 256
ROWS = 512
BIG = jnp.int32(1 << 30)
PACKN = 2048 + 8 * W + ROWS
I32 = jnp.int32


def _row_gather(src, idx, *, fill):
    n = src.shape[0]
    m = idx.shape[0]
    rest = src.shape[1:]

    def body(idx_ref, src_ref, out_ref):
        def step(i, _):
            j = idx_ref[i]
            if fill:
                @pl.when(j < n)
                def _():
                    out_ref[i] = src_ref[j]

                @pl.when(j >= n)
                def _():
                    out_ref[i] = jnp.zeros(rest, src_ref.dtype)
            else:
                out_ref[i] = src_ref[j]
            return 0

        lax.fori_loop(0, m, step, 0, unroll=8)

    return pl.pallas_call(
        body,
        out_shape=jax.ShapeDtypeStruct((m,) + rest, src.dtype),
        in_specs=[
            pl.BlockSpec(memory_space=pltpu.SMEM),
            pl.BlockSpec(memory_space=pltpu.VMEM),
        ],
        out_specs=pl.BlockSpec(memory_space=pltpu.VMEM),
    )(idx, src)


def _a2a(src, offs, rows, *, cid):
    rest = src.shape[1:]

    def body(offs_ref, src_ref, out_ref, send_sems, recv_sems):
        me = lax.axis_index("i")

        bar = pltpu.get_barrier_semaphore()
        for d in range(W):
            @pl.when(me != d)
            def _():
                pl.semaphore_signal(
                    bar, inc=1,
                    device_id=(d,), device_id_type=pl.DeviceIdType.MESH,
                )
        pl.semaphore_wait(bar, W - 1)

        for d in range(W):
            off = pl.multiple_of(offs_ref[d], 8)

            @pl.when(me == d)
            def _():
                out_ref[d] = src_ref[pl.ds(off, rows)]

            @pl.when(me != d)
            def _():
                pltpu.make_async_remote_copy(
                    src_ref=src_ref.at[pl.ds(off, rows)],
                    dst_ref=out_ref.at[me],
                    send_sem=send_sems.at[d],
                    recv_sem=recv_sems.at[me],
                    device_id=(d,),
                    device_id_type=pl.DeviceIdType.MESH,
                ).start()

        for s in range(W):
            @pl.when(me != s)
            def _():
                pltpu.make_async_remote_copy(
                    src_ref=src_ref.at[pl.ds(0, rows)],
                    dst_ref=out_ref.at[s],
                    send_sem=send_sems.at[s],
                    recv_sem=recv_sems.at[s],
                    device_id=(s,),
                    device_id_type=pl.DeviceIdType.MESH,
                ).wait_recv()

        for d in range(W):
            @pl.when(me != d)
            def _():
                pltpu.make_async_remote_copy(
                    src_ref=src_ref.at[pl.ds(0, rows)],
                    dst_ref=out_ref.at[d],
                    send_sem=send_sems.at[d],
                    recv_sem=recv_sems.at[d],
                    device_id=(d,),
                    device_id_type=pl.DeviceIdType.MESH,
                ).wait_send()

    return pl.pallas_call(
        body,
        out_shape=jax.ShapeDtypeStruct((W, rows) + rest, src.dtype),
        in_specs=[
            pl.BlockSpec(memory_space=pltpu.SMEM),
            pl.BlockSpec(memory_space=pltpu.VMEM),
        ],
        out_specs=pl.BlockSpec(memory_space=pltpu.VMEM),
        scratch_shapes=[
            pltpu.SemaphoreType.DMA((W,)),
            pltpu.SemaphoreType.DMA((W,)),
        ],
        compiler_params=pltpu.CompilerParams(collective_id=cid),
    )(offs, src)


def _moe_matmul(xin, expert_W):

    def body(x_ref, w_ref, o_ref):
        o_ref[...] = jnp.dot(
            x_ref[...], w_ref[0], preferred_element_type=jnp.float32
        )

    return pl.pallas_call(
        body,
        grid=(EL,),
        in_specs=[
            pl.BlockSpec((CST, D), lambda e: (e, 0)),
            pl.BlockSpec((1, D, H), lambda e: (e, 0, 0)),
        ],
        out_specs=pl.BlockSpec((CST, H), lambda e: (e, 0)),
        out_shape=jax.ShapeDtypeStruct((EL * CST, H), jnp.float32),
    )(xin, expert_W)


def _pack_gather_idx(grp):
    n = grp.shape[0]
    oh = grp[:, None] == jnp.arange(W + 1, dtype=grp.dtype)[None, :]
    within = (
        jnp.take_along_axis(
            jnp.cumsum(oh.astype(I32), axis=0), grp[:, None].astype(I32), axis=1
        )[:, 0]
        - 1
    )
    cnts = oh.sum(axis=0).astype(I32)[:W]
    acnts = ((cnts + 7) // 8) * 8
    aoffs = jnp.cumsum(acnts) - acnts
    aoffs_ext = jnp.concatenate([aoffs, jnp.array([BIG], I32)])
    pos = aoffs_ext[grp] + within
    idx = (
        jnp.zeros(PACKN, I32)
        .at[pos]
        .set(jnp.arange(n, dtype=I32), mode="drop")
    )
    return idx, aoffs


def kernel(x, router_W, route_idx, expert_W):
    del router_W
    me = lax.axis_index("i")

    rloc = route_idx.reshape(16, 128)
    route = _a2a(rloc, jnp.zeros((W,), I32), 16, cid=0).reshape(NT)

    perm = jnp.argsort(route, stable=True)
    sorted_e = route[perm]
    starts = jnp.searchsorted(sorted_e, jnp.arange(NE, dtype=sorted_e.dtype))
    rank_sorted = jnp.arange(NT, dtype=I32) - starts[sorted_e].astype(I32)
    rank = jnp.zeros(NT, I32).at[perm].set(rank_sorted)
    keep = rank < CAP
    gslot = jnp.where(keep, route * CST + rank, BIG)
    tok_of_gslot = (
        jnp.full(NE * CST, -1, I32)
        .at[gslot]
        .set(jnp.arange(NT, dtype=I32), mode="drop")
    )

    myroute = route_idx[:, 0]
    mykeep = lax.dynamic_slice(keep, (me * S,), (S,))
    dest = jnp.where(mykeep, myroute // EL, W).astype(I32)
    pidx, offs = _pack_gather_idx(dest)
    x_pack = _row_gather(x.reshape(S, 8, 128), pidx, fill=False)

    xrecv = _a2a(x_pack, offs, ROWS, cid=1)

    keep2 = keep.reshape(W, S)
    route2 = route.reshape(W, S)
    rank2 = rank.reshape(W, S)
    tome = keep2 & ((route2 // EL) == me)
    lslot = jnp.where(tome, (route2 - me * EL) * CST + rank2, BIG)
    j2 = jnp.cumsum(tome.astype(I32), axis=1) - 1
    src_row = jnp.arange(W, dtype=I32)[:, None] * ROWS + j2
    inv2 = (
        jnp.full(EL * CST, BIG, I32)
        .at[lslot.reshape(-1)]
        .set(src_row.reshape(-1), mode="drop")
    )
    xin3 = _row_gather(xrecv.reshape(W * ROWS, 8, 128), inv2, fill=True)
    xin = xin3.reshape(EL * CST, D)

    y = _moe_matmul(xin, expert_W)

    mytoks = lax.dynamic_slice(tok_of_gslot, (me * EL * CST,), (EL * CST,))
    cdest = jnp.where(mytoks >= 0, mytoks // S, W).astype(I32)
    cidx, coffs = _pack_gather_idx(cdest)
    y_pack = _row_gather(y.reshape(EL * CST, 8, 128), cidx, fill=False)

    yrecv = _a2a(y_pack, coffs, ROWS, cid=2)

    toks_by_s = tok_of_gslot.reshape(W, EL * CST)
    mine = (toks_by_s >= me * S) & (toks_by_s < (me + 1) * S)
    mytok = jnp.where(mine, toks_by_s - me * S, BIG)
    j3 = jnp.cumsum(mine.astype(I32), axis=1) - 1
    src_row3 = jnp.arange(W, dtype=I32)[:, None] * ROWS + j3
    inv4 = (
        jnp.full(S, BIG, I32)
        .at[mytok.reshape(-1)]
        .set(src_row3.reshape(-1), mode="drop")
    )
    out3 = _row_gather(yrecv.reshape(W * ROWS, 8, 128), inv4, fill=True)
    return out3.reshape(S, H)


# device time: 422600 ns/iter; 8.4562x vs baseline; 1.7493x over previous
import jax
import jax.numpy as jnp
from jax import lax
from jax.experimental import pallas as pl
from jax.experimental.pallas import tpu as pltpu

W = 8
NT = 16384
S = NT // W
D = 1024
H = 1024
NE = 64
EL = NE // W
CAP = 204
CST = 256
ROWS = 512
BIG = jnp.int32(1 << 30)
PACKN = S + ROWS
I32 = jnp.int32
F32 = jnp.float32


def _a2a(src, offs, rows, *, cid):
    rest = src.shape[1:]

    def body(offs_ref, src_ref, out_ref, send_sems, recv_sems):
        me = lax.axis_index("i")

        bar = pltpu.get_barrier_semaphore()
        for d in range(W):
            @pl.when(me != d)
            def _():
                pl.semaphore_signal(
                    bar, inc=1,
                    device_id=(d,), device_id_type=pl.DeviceIdType.MESH,
                )
        pl.semaphore_wait(bar, W - 1)

        for d in range(W):
            off = offs_ref[d]

            @pl.when(me == d)
            def _():
                out_ref[d] = src_ref[pl.ds(off, rows)]

            @pl.when(me != d)
            def _():
                pltpu.make_async_remote_copy(
                    src_ref=src_ref.at[pl.ds(off, rows)],
                    dst_ref=out_ref.at[me],
                    send_sem=send_sems.at[d],
                    recv_sem=recv_sems.at[me],
                    device_id=(d,),
                    device_id_type=pl.DeviceIdType.MESH,
                ).start()

        for s in range(W):
            @pl.when(me != s)
            def _():
                pltpu.make_async_remote_copy(
                    src_ref=src_ref.at[pl.ds(0, rows)],
                    dst_ref=out_ref.at[s],
                    send_sem=send_sems.at[s],
                    recv_sem=recv_sems.at[s],
                    device_id=(s,),
                    device_id_type=pl.DeviceIdType.MESH,
                ).wait_recv()

        for d in range(W):
            @pl.when(me != d)
            def _():
                pltpu.make_async_remote_copy(
                    src_ref=src_ref.at[pl.ds(0, rows)],
                    dst_ref=out_ref.at[d],
                    send_sem=send_sems.at[d],
                    recv_sem=recv_sems.at[d],
                    device_id=(d,),
                    device_id_type=pl.DeviceIdType.MESH,
                ).wait_send()

    return pl.pallas_call(
        body,
        out_shape=jax.ShapeDtypeStruct((W, rows) + rest, src.dtype),
        in_specs=[
            pl.BlockSpec(memory_space=pltpu.SMEM),
            pl.BlockSpec(memory_space=pltpu.VMEM),
        ],
        out_specs=pl.BlockSpec(memory_space=pltpu.VMEM),
        scratch_shapes=[
            pltpu.SemaphoreType.DMA((W,)),
            pltpu.SemaphoreType.DMA((W,)),
        ],
        compiler_params=pltpu.CompilerParams(collective_id=cid),
    )(offs, src)


def _row_gather(src, idx, *, fill):
    n = src.shape[0]
    m = idx.shape[0]
    rest = src.shape[1:]

    def body(idx_ref, src_ref, out_ref):
        def step(i, _):
            j = idx_ref[i]
            if fill:
                @pl.when(j < n)
                def _():
                    out_ref[i] = src_ref[j]

                @pl.when(j >= n)
                def _():
                    out_ref[i] = jnp.zeros(rest, src_ref.dtype)
            else:
                out_ref[i] = src_ref[j]
            return 0

        lax.fori_loop(0, m, step, 0, unroll=8)

    return pl.pallas_call(
        body,
        out_shape=jax.ShapeDtypeStruct((m,) + rest, src.dtype),
        in_specs=[
            pl.BlockSpec(memory_space=pltpu.SMEM),
            pl.BlockSpec(memory_space=pltpu.VMEM),
        ],
        out_specs=pl.BlockSpec(memory_space=pltpu.VMEM),
    )(idx, src)


def _row_scatter(src, pos, out_rows):
    m = src.shape[0]
    rest = src.shape[1:]

    def body(pos_ref, src_ref, out_ref):
        def step(i, _):
            p = pos_ref[i]

            @pl.when(p < out_rows)
            def _():
                out_ref[p] = src_ref[i]

            return 0

        lax.fori_loop(0, m, step, 0, unroll=8)

    return pl.pallas_call(
        body,
        out_shape=jax.ShapeDtypeStruct((out_rows,) + rest, src.dtype),
        in_specs=[
            pl.BlockSpec(memory_space=pltpu.SMEM),
            pl.BlockSpec(memory_space=pltpu.VMEM),
        ],
        out_specs=pl.BlockSpec(memory_space=pltpu.VMEM),
    )(pos, src)


def _moe_matmul(xin, expert_W):

    def body(x_ref, w_ref, o_ref):
        o_ref[...] = jnp.dot(
            x_ref[...], w_ref[0], preferred_element_type=F32
        )

    return pl.pallas_call(
        body,
        grid=(EL,),
        in_specs=[
            pl.BlockSpec((CST, D), lambda e: (e, 0)),
            pl.BlockSpec((1, D, H), lambda e: (e, 0, 0)),
        ],
        out_specs=pl.BlockSpec((CST, H), lambda e: (e, 0)),
        out_shape=jax.ShapeDtypeStruct((EL * CST, H), F32),
    )(xin, expert_W)


def _onehot_pick(oh, table):
    return (oh[:, : table.shape[0]] * table.astype(F32)[None, :]).sum(-1)


def kernel(x, router_W, route_idx, expert_W):
    del router_W
    me = lax.axis_index("i")

    route = _a2a(
        route_idx.reshape(2, 8, 128), jnp.zeros((W,), I32), 2, cid=0
    ).reshape(NT)

    tril128 = jnp.tril(jnp.ones((128, 128), F32), -1)
    oh = (
        route.reshape(128, 128)[:, :, None]
        == jnp.arange(NE, dtype=route.dtype)[None, None, :]
    ).astype(F32)
    intra = jnp.einsum(
        "pq,bqe->bpe", tril128, oh, preferred_element_type=F32
    )
    blocksum = oh.sum(axis=1)
    blockpref = tril128 @ blocksum
    rank = (
        (oh * (intra + blockpref[:, None, :])).sum(-1).reshape(NT).astype(I32)
    )
    keep = rank < CAP
    gslot = jnp.where(keep, route * CST + rank, BIG)

    shard_hist = blocksum.reshape(W, 16, NE).sum(1)
    cum_shard = jnp.tril(jnp.ones((W, W), F32), -1) @ shard_hist
    total_e = shard_hist.sum(0)

    myroute = route_idx[:, 0]
    mygslot = lax.dynamic_slice(gslot, (me * S,), (S,))
    mykeep = lax.dynamic_slice(rank, (me * S,), (S,)) < CAP
    dest = jnp.where(mykeep, myroute // EL, W).astype(I32)
    ohD = (dest[:, None] == jnp.arange(W + 1, dtype=I32)[None, :]).astype(F32)
    cmp = (mygslot[None, :] < mygslot[:, None]).astype(F32)
    within_tok = (
        (ohD * jnp.einsum("lt,td->ld", cmp, ohD, preferred_element_type=F32))
        .sum(-1)
        .astype(I32)
    )
    cntsD = ohD.sum(0).astype(I32)[:W]
    offsD = (jnp.cumsum(cntsD) - cntsD).astype(I32)
    posD = jnp.where(
        dest < W, _onehot_pick(ohD, offsD).astype(I32) + within_tok, BIG
    )
    rowidx_tok = jnp.where(dest < W, dest * ROWS + within_tok, BIG)

    cums_my = lax.dynamic_slice(cum_shard, (0, me * EL), (W, EL))
    tot_my = lax.dynamic_slice(total_e, (me * EL,), (EL,))
    r_grid = jnp.arange(CST, dtype=I32)[None, None, :]
    csrc_raw = (
        (r_grid >= cums_my.astype(I32)[:, :, None]).sum(0).astype(I32) - 1
    )
    validslot = r_grid[0] < jnp.minimum(tot_my, float(CAP))[:, None].astype(I32)
    csrc = jnp.where(validslot, csrc_raw, W).reshape(S).astype(I32)
    ohC = (csrc[:, None] == jnp.arange(W + 1, dtype=I32)[None, :]).astype(F32)
    tril_s = jnp.tril(jnp.ones((S, S), F32), -1)
    within_slot = (
        (ohC * jnp.einsum("lt,td->ld", tril_s, ohC, preferred_element_type=F32))
        .sum(-1)
        .astype(I32)
    )
    cntsC = ohC.sum(0).astype(I32)[:W]
    offsC = (jnp.cumsum(cntsC) - cntsC).astype(I32)
    rowidx_slot = jnp.where(csrc < W, csrc * ROWS + within_slot, BIG)
    posC = jnp.where(
        csrc < W, _onehot_pick(ohC, offsC).astype(I32) + within_slot, BIG
    )

    x_pack = _row_scatter(x.reshape(S, 8, 128), posD, PACKN)
    xrecv = _a2a(x_pack, offsD, ROWS, cid=1)
    xin3 = _row_gather(xrecv.reshape(W * ROWS, 8, 128), rowidx_slot, fill=True)

    y = _moe_matmul(xin3.reshape(EL * CST, D), expert_W)

    y_pack = _row_scatter(y.reshape(S, 8, 128), posC, PACKN)
    yrecv = _a2a(y_pack, offsC, ROWS, cid=2)
    out3 = _row_gather(yrecv.reshape(W * ROWS, 8, 128), rowidx_tok, fill=True)
    return out3.reshape(S, H)


# device time: 275341 ns/iter; 12.9787x vs baseline; 1.5348x over previous
import jax
import jax.numpy as jnp
from jax import lax
from jax.experimental import pallas as pl
from jax.experimental.pallas import tpu as pltpu

W = 8
NT = 16384
S = NT // W
D = 1024
H = 1024
NE = 64
EL = NE // W
CAP = 204
CST = 256
ROWS = 384
BIG = jnp.int32(1 << 30)
PACKN = S + ROWS
I32 = jnp.int32
F32 = jnp.float32


def _a2a(src, offs, rows, *, cid):
    rest = src.shape[1:]

    def body(offs_ref, src_ref, out_ref, send_sems, recv_sems):
        me = lax.axis_index("i")

        bar = pltpu.get_barrier_semaphore()
        for d in range(W):
            @pl.when(me != d)
            def _():
                pl.semaphore_signal(
                    bar, inc=1,
                    device_id=(d,), device_id_type=pl.DeviceIdType.MESH,
                )
        pl.semaphore_wait(bar, W - 1)

        for d in range(W):
            off = offs_ref[d]

            @pl.when(me == d)
            def _():
                out_ref[d] = src_ref[pl.ds(off, rows)]

            @pl.when(me != d)
            def _():
                pltpu.make_async_remote_copy(
                    src_ref=src_ref.at[pl.ds(off, rows)],
                    dst_ref=out_ref.at[me],
                    send_sem=send_sems.at[d],
                    recv_sem=recv_sems.at[me],
                    device_id=(d,),
                    device_id_type=pl.DeviceIdType.MESH,
                ).start()

        for s in range(W):
            @pl.when(me != s)
            def _():
                pltpu.make_async_remote_copy(
                    src_ref=src_ref.at[pl.ds(0, rows)],
                    dst_ref=out_ref.at[s],
                    send_sem=send_sems.at[s],
                    recv_sem=recv_sems.at[s],
                    device_id=(s,),
                    device_id_type=pl.DeviceIdType.MESH,
                ).wait_recv()

        for d in range(W):
            @pl.when(me != d)
            def _():
                pltpu.make_async_remote_copy(
                    src_ref=src_ref.at[pl.ds(0, rows)],
                    dst_ref=out_ref.at[d],
                    send_sem=send_sems.at[d],
                    recv_sem=recv_sems.at[d],
                    device_id=(d,),
                    device_id_type=pl.DeviceIdType.MESH,
                ).wait_send()

    return pl.pallas_call(
        body,
        out_shape=jax.ShapeDtypeStruct((W, rows) + rest, src.dtype),
        in_specs=[
            pl.BlockSpec(memory_space=pltpu.SMEM),
            pl.BlockSpec(memory_space=pltpu.VMEM),
        ],
        out_specs=pl.BlockSpec(memory_space=pltpu.VMEM),
        scratch_shapes=[
            pltpu.SemaphoreType.DMA((W,)),
            pltpu.SemaphoreType.DMA((W,)),
        ],
        compiler_params=pltpu.CompilerParams(collective_id=cid),
    )(offs, src)


def _row_gather(src, idx):
    m = idx.shape[0]
    rest = src.shape[1:]

    def body(idx_ref, src_ref, out_ref):
        def step(i, _):
            out_ref[i] = src_ref[idx_ref[i]]
            return 0

        lax.fori_loop(0, m, step, 0, unroll=8)

    return pl.pallas_call(
        body,
        out_shape=jax.ShapeDtypeStruct((m,) + rest, src.dtype),
        in_specs=[
            pl.BlockSpec(memory_space=pltpu.SMEM),
            pl.BlockSpec(memory_space=pltpu.VMEM),
        ],
        out_specs=pl.BlockSpec(memory_space=pltpu.VMEM),
    )(idx, src)


def _row_scatter(src, pos, out_rows):
    m = src.shape[0]
    rest = src.shape[1:]

    def body(pos_ref, src_ref, out_ref):
        def step(i, _):
            out_ref[pos_ref[i]] = src_ref[i]
            return 0

        lax.fori_loop(0, m, step, 0, unroll=8)

    return pl.pallas_call(
        body,
        out_shape=jax.ShapeDtypeStruct((out_rows,) + rest, src.dtype),
        in_specs=[
            pl.BlockSpec(memory_space=pltpu.SMEM),
            pl.BlockSpec(memory_space=pltpu.VMEM),
        ],
        out_specs=pl.BlockSpec(memory_space=pltpu.VMEM),
    )(pos, src)


def _moe_matmul(xin, expert_W):

    def body(x_ref, w_ref, o_ref):
        o_ref[...] = jnp.dot(
            x_ref[...], w_ref[0], preferred_element_type=F32
        )

    return pl.pallas_call(
        body,
        grid=(EL,),
        in_specs=[
            pl.BlockSpec((CST, D), lambda e: (e, 0)),
            pl.BlockSpec((1, D, H), lambda e: (e, 0, 0)),
        ],
        out_specs=pl.BlockSpec((CST, H), lambda e: (e, 0)),
        out_shape=jax.ShapeDtypeStruct((EL * CST, H), F32),
    )(xin, expert_W)


def _onehot_pick(oh, table):
    return (oh[:, : table.shape[0]] * table.astype(F32)[None, :]).sum(-1)


def kernel(x, router_W, route_idx, expert_W):
    del router_W
    me = lax.axis_index("i")

    route = _a2a(
        route_idx.reshape(2, 8, 128), jnp.zeros((W,), I32), 2, cid=0
    ).reshape(NT)

    tril128 = jnp.tril(jnp.ones((128, 128), F32), -1)
    oh = (
        route.reshape(128, 128)[:, :, None]
        == jnp.arange(NE, dtype=route.dtype)[None, None, :]
    ).astype(F32)
    intra = jnp.einsum(
        "pq,bqe->bpe", tril128, oh, preferred_element_type=F32
    )
    blocksum = oh.sum(axis=1)
    blockpref = tril128 @ blocksum
    rank = (
        (oh * (intra + blockpref[:, None, :])).sum(-1).reshape(NT).astype(I32)
    )
    keep = rank < CAP
    gslot = jnp.where(keep, route * CST + rank, BIG)

    shard_hist = blocksum.reshape(W, 16, NE).sum(1)
    cum_shard = jnp.tril(jnp.ones((W, W), F32), -1) @ shard_hist
    total_e = shard_hist.sum(0)

    myroute = route_idx[:, 0]
    mygslot = lax.dynamic_slice(gslot, (me * S,), (S,))
    mykeep = lax.dynamic_slice(rank, (me * S,), (S,)) < CAP
    dest = jnp.where(mykeep, myroute // EL, W).astype(I32)
    ohD = (dest[:, None] == jnp.arange(W + 1, dtype=I32)[None, :]).astype(F32)
    cmp = (mygslot[None, :] < mygslot[:, None]).astype(F32)
    within_tok = (
        (ohD * jnp.einsum("lt,td->ld", cmp, ohD, preferred_element_type=F32))
        .sum(-1)
        .astype(I32)
    )
    cntsD = ohD.sum(0).astype(I32)[:W]
    offsD = (jnp.cumsum(cntsD) - cntsD).astype(I32)
    posD = jnp.where(
        dest < W, _onehot_pick(ohD, offsD).astype(I32) + within_tok, PACKN - 1
    )
    rowidx_tok = jnp.where(dest < W, dest * ROWS + within_tok, 0)

    cums_my = lax.dynamic_slice(cum_shard, (0, me * EL), (W, EL))
    tot_my = lax.dynamic_slice(total_e, (me * EL,), (EL,))
    r_grid = jnp.arange(CST, dtype=I32)[None, None, :]
    csrc_raw = (
        (r_grid >= cums_my.astype(I32)[:, :, None]).sum(0).astype(I32) - 1
    )
    validslot = r_grid[0] < jnp.minimum(tot_my, float(CAP))[:, None].astype(I32)
    csrc = jnp.where(validslot, csrc_raw, W).reshape(S).astype(I32)
    ohC = (csrc[:, None] == jnp.arange(W + 1, dtype=I32)[None, :]).astype(F32)
    tril_s = jnp.tril(jnp.ones((S, S), F32), -1)
    within_slot = (
        (ohC * jnp.einsum("lt,td->ld", tril_s, ohC, preferred_element_type=F32))
        .sum(-1)
        .astype(I32)
    )
    cntsC = ohC.sum(0).astype(I32)[:W]
    offsC = (jnp.cumsum(cntsC) - cntsC).astype(I32)
    rowidx_slot = jnp.where(csrc < W, csrc * ROWS + within_slot, 0)
    posC = jnp.where(
        csrc < W, _onehot_pick(ohC, offsC).astype(I32) + within_slot, PACKN - 1
    )

    x_pack = _row_scatter(x.reshape(S, 8, 128), posD, PACKN)
    xrecv = _a2a(x_pack, offsD, ROWS, cid=1)
    xin3 = _row_gather(xrecv.reshape(W * ROWS, 8, 128), rowidx_slot)

    y = _moe_matmul(xin3.reshape(EL * CST, D), expert_W)

    y_pack = _row_scatter(y.reshape(S, 8, 128), posC, PACKN)
    yrecv = _a2a(y_pack, offsC, ROWS, cid=2)
    out3 = _row_gather(yrecv.reshape(W * ROWS, 8, 128), rowidx_tok)
    return jnp.where((dest < W)[:, None], out3.reshape(S, H), 0.0)


# device time: 237889 ns/iter; 15.0220x vs baseline; 1.1574x over previous
import jax
import jax.numpy as jnp
from jax import lax
from jax.experimental import pallas as pl
from jax.experimental.pallas import tpu as pltpu

W = 8
NT = 16384
S = NT // W
D = 1024
H = 1024
NE = 64
EL = NE // W
CAP = 204
CST = 256
ROWS = 384
BIG = jnp.int32(1 << 30)
PACKN = S + ROWS
I32 = jnp.int32
F32 = jnp.float32


def _a2a(src, offs, rows, cnt_send, cnt_recv, chunk, *, cid):
    rest = src.shape[1:]
    nc = (rows + chunk - 1) // chunk

    def body(offs_ref, cs_ref, cr_ref, src_ref, out_ref, send_sems, recv_sems):
        me = lax.axis_index("i")

        bar = pltpu.get_barrier_semaphore()
        for d in range(W):
            @pl.when(me != d)
            def _():
                pl.semaphore_signal(
                    bar, inc=1,
                    device_id=(d,), device_id_type=pl.DeviceIdType.MESH,
                )
        pl.semaphore_wait(bar, W - 1)

        for d in range(W):
            off = offs_ref[d]

            @pl.when(me == d)
            def _():
                out_ref[d] = src_ref[pl.ds(off, rows)]

            for c in range(nc):
                @pl.when((me != d) & (cs_ref[d] > c * chunk))
                def _():
                    pltpu.make_async_remote_copy(
                        src_ref=src_ref.at[pl.ds(off + c * chunk, chunk)],
                        dst_ref=out_ref.at[me, pl.ds(c * chunk, chunk)],
                        send_sem=send_sems.at[d, c],
                        recv_sem=recv_sems.at[me, c],
                        device_id=(d,),
                        device_id_type=pl.DeviceIdType.MESH,
                    ).start()

        for s in range(W):
            for c in range(nc):
                @pl.when((me != s) & (cr_ref[s] > c * chunk))
                def _():
                    pltpu.make_async_remote_copy(
                        src_ref=src_ref.at[pl.ds(0, chunk)],
                        dst_ref=out_ref.at[s, pl.ds(c * chunk, chunk)],
                        send_sem=send_sems.at[s, c],
                        recv_sem=recv_sems.at[s, c],
                        device_id=(s,),
                        device_id_type=pl.DeviceIdType.MESH,
                    ).wait_recv()

        for d in range(W):
            for c in range(nc):
                @pl.when((me != d) & (cs_ref[d] > c * chunk))
                def _():
                    pltpu.make_async_remote_copy(
                        src_ref=src_ref.at[pl.ds(0, chunk)],
                        dst_ref=out_ref.at[d, pl.ds(c * chunk, chunk)],
                        send_sem=send_sems.at[d, c],
                        recv_sem=recv_sems.at[d, c],
                        device_id=(d,),
                        device_id_type=pl.DeviceIdType.MESH,
                    ).wait_send()

    return pl.pallas_call(
        body,
        out_shape=jax.ShapeDtypeStruct((W, rows) + rest, src.dtype),
        in_specs=[
            pl.BlockSpec(memory_space=pltpu.SMEM),
            pl.BlockSpec(memory_space=pltpu.SMEM),
            pl.BlockSpec(memory_space=pltpu.SMEM),
            pl.BlockSpec(memory_space=pltpu.VMEM),
        ],
        out_specs=pl.BlockSpec(memory_space=pltpu.VMEM),
        scratch_shapes=[
            pltpu.SemaphoreType.DMA((W, nc)),
            pltpu.SemaphoreType.DMA((W, nc)),
        ],
        compiler_params=pltpu.CompilerParams(collective_id=cid),
    )(offs, cnt_send, cnt_recv, src)


def _row_gather(src, idx):
    m = idx.shape[0]
    rest = src.shape[1:]

    def body(idx_ref, src_ref, out_ref):
        def step(i, _):
            out_ref[i] = src_ref[idx_ref[i]]
            return 0

        lax.fori_loop(0, m, step, 0, unroll=8)

    return pl.pallas_call(
        body,
        out_shape=jax.ShapeDtypeStruct((m,) + rest, src.dtype),
        in_specs=[
            pl.BlockSpec(memory_space=pltpu.SMEM),
            pl.BlockSpec(memory_space=pltpu.VMEM),
        ],
        out_specs=pl.BlockSpec(memory_space=pltpu.VMEM),
    )(idx, src)


def _row_scatter(src, pos, out_rows):
    m = src.shape[0]
    rest = src.shape[1:]

    def body(pos_ref, src_ref, out_ref):
        def step(i, _):
            out_ref[pos_ref[i]] = src_ref[i]
            return 0

        lax.fori_loop(0, m, step, 0, unroll=8)

    return pl.pallas_call(
        body,
        out_shape=jax.ShapeDtypeStruct((out_rows,) + rest, src.dtype),
        in_specs=[
            pl.BlockSpec(memory_space=pltpu.SMEM),
            pl.BlockSpec(memory_space=pltpu.VMEM),
        ],
        out_specs=pl.BlockSpec(memory_space=pltpu.VMEM),
    )(pos, src)


def _moe_matmul(xin, expert_W):

    def body(x_ref, w_ref, o_ref):
        o_ref[...] = jnp.dot(
            x_ref[...], w_ref[0], preferred_element_type=F32
        )

    return pl.pallas_call(
        body,
        grid=(EL,),
        in_specs=[
            pl.BlockSpec((CST, D), lambda e: (e, 0)),
            pl.BlockSpec((1, D, H), lambda e: (e, 0, 0)),
        ],
        out_specs=pl.BlockSpec((CST, H), lambda e: (e, 0)),
        out_shape=jax.ShapeDtypeStruct((EL * CST, H), F32),
    )(xin, expert_W)


def _onehot_pick(oh, table):
    return (oh[:, : table.shape[0]] * table.astype(F32)[None, :]).sum(-1)


def kernel(x, router_W, route_idx, expert_W):
    del router_W
    me = lax.axis_index("i")

    full2 = jnp.full((W,), 2, I32)
    route = _a2a(
        route_idx.reshape(2, 8, 128), jnp.zeros((W,), I32), 2,
        full2, full2, 2, cid=0,
    ).reshape(NT)

    tril128 = jnp.tril(jnp.ones((128, 128), F32), -1)
    oh = (
        route.reshape(128, 128)[:, :, None]
        == jnp.arange(NE, dtype=route.dtype)[None, None, :]
    ).astype(F32)
    intra = jnp.einsum(
        "pq,bqe->bpe", tril128, oh, preferred_element_type=F32
    )
    blocksum = oh.sum(axis=1)
    blockpref = tril128 @ blocksum
    rank = (
        (oh * (intra + blockpref[:, None, :])).sum(-1).reshape(NT).astype(I32)
    )
    keep = rank < CAP
    gslot = jnp.where(keep, route * CST + rank, BIG)

    shard_hist = blocksum.reshape(W, 16, NE).sum(1)
    cum_shard = jnp.tril(jnp.ones((W, W), F32), -1) @ shard_hist
    total_e = shard_hist.sum(0)

    myroute = route_idx[:, 0]
    mygslot = lax.dynamic_slice(gslot, (me * S,), (S,))
    mykeep = lax.dynamic_slice(rank, (me * S,), (S,)) < CAP
    dest = jnp.where(mykeep, myroute // EL, W).astype(I32)
    ohD = (dest[:, None] == jnp.arange(W + 1, dtype=I32)[None, :]).astype(F32)
    cmp = (mygslot[None, :] < mygslot[:, None]).astype(F32)
    within_tok = (
        (ohD * jnp.einsum("lt,td->ld", cmp, ohD, preferred_element_type=F32))
        .sum(-1)
        .astype(I32)
    )
    cntsD = ohD.sum(0).astype(I32)[:W]
    offsD = (jnp.cumsum(cntsD) - cntsD).astype(I32)
    posD = jnp.where(
        dest < W, _onehot_pick(ohD, offsD).astype(I32) + within_tok, PACKN - 1
    )
    rowidx_tok = jnp.where(dest < W, dest * ROWS + within_tok, 0)

    cums_my = lax.dynamic_slice(cum_shard, (0, me * EL), (W, EL))
    tot_my = lax.dynamic_slice(total_e, (me * EL,), (EL,))
    r_grid = jnp.arange(CST, dtype=I32)[None, None, :]
    csrc_raw = (
        (r_grid >= cums_my.astype(I32)[:, :, None]).sum(0).astype(I32) - 1
    )
    validslot = r_grid[0] < jnp.minimum(tot_my, float(CAP))[:, None].astype(I32)
    csrc = jnp.where(validslot, csrc_raw, W).reshape(S).astype(I32)
    ohC = (csrc[:, None] == jnp.arange(W + 1, dtype=I32)[None, :]).astype(F32)
    tril_s = jnp.tril(jnp.ones((S, S), F32), -1)
    within_slot = (
        (ohC * jnp.einsum("lt,td->ld", tril_s, ohC, preferred_element_type=F32))
        .sum(-1)
        .astype(I32)
    )
    cntsC = ohC.sum(0).astype(I32)[:W]
    offsC = (jnp.cumsum(cntsC) - cntsC).astype(I32)
    rowidx_slot = jnp.where(csrc < W, csrc * ROWS + within_slot, 0)
    posC = jnp.where(
        csrc < W, _onehot_pick(ohC, offsC).astype(I32) + within_slot, PACKN - 1
    )

    x_pack = _row_scatter(x.reshape(S, 8, 128), posD, PACKN)
    xrecv = _a2a(x_pack, offsD, ROWS, cntsD, cntsC, 128, cid=1)
    xin3 = _row_gather(xrecv.reshape(W * ROWS, 8, 128), rowidx_slot)

    y = _moe_matmul(xin3.reshape(EL * CST, D), expert_W)

    y_pack = _row_scatter(y.reshape(S, 8, 128), posC, PACKN)
    yrecv = _a2a(y_pack, offsC, ROWS, cntsC, cntsD, 128, cid=2)
    out3 = _row_gather(yrecv.reshape(W * ROWS, 8, 128), rowidx_tok)
    return jnp.where((dest < W)[:, None], out3.reshape(S, H), 0.0)
